# baseline (speedup 1.0000x reference)
import functools

import jax
import jax.numpy as jnp
from jax import lax
from jax.experimental import pallas as pl
from jax.experimental.pallas import tpu as pltpu
from jax.experimental.pallas import tpu_sc as plsc

N = 10000
D = 128
E = 320000
G = 64

NC = 2
NS = 16
NW = NC * NS
CH = 128
IB = 16
NB = 5
KPW = NB * IB
EPW = KPW * CH
E_PAD = NW * EPW
N_PAD = 10240
N_CHUNKS = N_PAD // CH
RC_SUB = N_CHUNKS // NS

_mesh = plsc.VectorSubcoreMesh(core_axis_name="c", subcore_axis_name="s")


@functools.partial(
    pl.kernel,
    out_type=jax.ShapeDtypeStruct((NC, N_PAD, D), jnp.float32),
    mesh=_mesh,
    scratch_types=[
        pltpu.VMEM((2, IB, CH), jnp.int32),
        pltpu.VMEM((2, IB, CH), jnp.int32),
        pltpu.VMEM((CH, D), jnp.float32),
        pltpu.VMEM((CH, D), jnp.float32),
        pltpu.VMEM_SHARED((N_PAD, D), jnp.float32),
        pltpu.SemaphoreType.DMA,
        pltpu.SemaphoreType.DMA,
        pltpu.SemaphoreType.DMA,
    ],
)
def _sc_agg(h_hbm, src_hbm, dst_hbm, out_hbm, src_v, dst_v, rows_a, rows_b,
            agg_sh, sem_a, sem_b, sem_i):
    cid = lax.axis_index("c")
    sid = lax.axis_index("s")
    wid = cid * NS + sid

    pltpu.async_copy(src_hbm.at[wid, 0], src_v.at[0], sem_i)
    pltpu.async_copy(dst_hbm.at[wid, 0], dst_v.at[0], sem_i)

    def _zero_row(i, carry):
        for j in range(D // 16):
            rows_a[i, pl.ds(j * 16, 16)] = jnp.zeros((16,), jnp.float32)
        return carry

    lax.fori_loop(0, CH, _zero_row, 0)

    def _zero_chunk(z, carry):
        c = sid * RC_SUB + z
        pltpu.sync_copy(rows_a, agg_sh.at[pl.ds(c * CH, CH)])
        return carry

    lax.fori_loop(0, RC_SUB, _zero_chunk, 0)

    pltpu.make_async_copy(src_hbm.at[wid, 0], src_v.at[0], sem_i).wait()
    pltpu.make_async_copy(dst_hbm.at[wid, 0], dst_v.at[0], sem_i).wait()

    plsc.subcore_barrier()

    for b in range(NB):
        slot = b % 2
        if b + 1 < NB:
            pltpu.async_copy(src_hbm.at[wid, b + 1], src_v.at[1 - slot], sem_i)
            pltpu.async_copy(dst_hbm.at[wid, b + 1], dst_v.at[1 - slot], sem_i)
        pltpu.async_copy(h_hbm.at[src_v.at[slot, 0]], rows_a, sem_a)

        def _pair(p, carry, slot=slot):
            j = 2 * p
            pltpu.async_copy(h_hbm.at[src_v.at[slot, j + 1]], rows_b, sem_b)
            pltpu.make_async_copy(h_hbm.at[src_v.at[slot, j]], rows_a, sem_a).wait()
            pltpu.sync_copy(rows_a, agg_sh.at[dst_v.at[slot, j]], add=True)
            jn = lax.rem(j + 2, IB)
            pltpu.async_copy(h_hbm.at[src_v.at[slot, jn]], rows_a, sem_a)
            pltpu.make_async_copy(h_hbm.at[src_v.at[slot, j + 1]], rows_b, sem_b).wait()
            pltpu.sync_copy(rows_b, agg_sh.at[dst_v.at[slot, j + 1]], add=True)
            return carry

        lax.fori_loop(0, IB // 2, _pair, 0)
        pltpu.make_async_copy(h_hbm.at[src_v.at[slot, 0]], rows_a, sem_a).wait()
        if b + 1 < NB:
            pltpu.make_async_copy(src_hbm.at[wid, b + 1], src_v.at[1 - slot], sem_i).wait()
            pltpu.make_async_copy(dst_hbm.at[wid, b + 1], dst_v.at[1 - slot], sem_i).wait()

    plsc.subcore_barrier()

    def _export_chunk(z, carry):
        c = sid * RC_SUB + z
        pltpu.sync_copy(agg_sh.at[pl.ds(c * CH, CH)], rows_a)
        pltpu.sync_copy(rows_a, out_hbm.at[cid, pl.ds(c * CH, CH)])
        return carry

    lax.fori_loop(0, RC_SUB, _export_chunk, 0)


BLK = 1024


def _mlp_body(h_ref, a_ref, w1_ref, b1_ref, w2_ref, b2_ref, o_ref):
    z = h_ref[...] + a_ref[0] + a_ref[1]
    z = jnp.maximum(
        jnp.dot(z, w1_ref[...], preferred_element_type=jnp.float32) + b1_ref[...],
        0.0,
    )
    z = jnp.dot(z, w2_ref[...], preferred_element_type=jnp.float32) + b2_ref[...]
    o_ref[...] = jnp.maximum(z, 0.0)


_mlp = pl.pallas_call(
    _mlp_body,
    grid=(N_PAD // BLK,),
    in_specs=[
        pl.BlockSpec((BLK, D), lambda i: (i, 0)),
        pl.BlockSpec((NC, BLK, D), lambda i: (0, i, 0)),
        pl.BlockSpec((D, D), lambda i: (0, 0)),
        pl.BlockSpec((1, D), lambda i: (0, 0)),
        pl.BlockSpec((D, D), lambda i: (0, 0)),
        pl.BlockSpec((1, D), lambda i: (0, 0)),
    ],
    out_specs=pl.BlockSpec((BLK, D), lambda i: (i, 0)),
    out_shape=jax.ShapeDtypeStruct((N_PAD, D), jnp.float32),
)


def _mlp_pool_body(h_ref, a_ref, w1_ref, b1_ref, w2_ref, b2_ref, bt_ref, o_ref):
    i = pl.program_id(0)
    z = h_ref[...] + a_ref[0] + a_ref[1]
    z = jnp.maximum(
        jnp.dot(z, w1_ref[...], preferred_element_type=jnp.float32) + b1_ref[...],
        0.0,
    )
    z = jnp.dot(z, w2_ref[...], preferred_element_type=jnp.float32) + b2_ref[...]
    z = jnp.maximum(z, 0.0)
    onehot = (lax.broadcasted_iota(jnp.int32, (BLK, G), 1) == bt_ref[...]).astype(
        jnp.float32
    )
    contrib = lax.dot_general(
        onehot, z, (((0,), (0,)), ((), ())), preferred_element_type=jnp.float32
    )

    @pl.when(i == 0)
    def _():
        o_ref[...] = jnp.zeros_like(o_ref)

    o_ref[...] += contrib


_mlp_pool = pl.pallas_call(
    _mlp_pool_body,
    grid=(N_PAD // BLK,),
    in_specs=[
        pl.BlockSpec((BLK, D), lambda i: (i, 0)),
        pl.BlockSpec((NC, BLK, D), lambda i: (0, i, 0)),
        pl.BlockSpec((D, D), lambda i: (0, 0)),
        pl.BlockSpec((1, D), lambda i: (0, 0)),
        pl.BlockSpec((D, D), lambda i: (0, 0)),
        pl.BlockSpec((1, D), lambda i: (0, 0)),
        pl.BlockSpec((BLK, 1), lambda i: (i, 0)),
    ],
    out_specs=pl.BlockSpec((G, D), lambda i: (0, 0)),
    out_shape=jax.ShapeDtypeStruct((G, D), jnp.float32),
)


def kernel(x, edge_index, batch, W1_0, b1_0, W2_0, b2_0, W1_1, b1_1, W2_1, b2_1,
           W1_2, b1_2, W2_2, b2_2):
    src = edge_index[0]
    dst = edge_index[1]
    pad_e = E_PAD - E
    src_p = jnp.concatenate([src, jnp.zeros((pad_e,), jnp.int32)]).reshape(NW, NB, IB, CH)
    dst_p = jnp.concatenate([dst, jnp.full((pad_e,), N, jnp.int32)]).reshape(NW, NB, IB, CH)
    h = jnp.pad(x, ((0, N_PAD - N), (0, 0)))
    bt = jnp.concatenate([batch, jnp.full((N_PAD - N,), G, jnp.int32)]).reshape(N_PAD, 1)

    params = [
        (W1_0, b1_0.reshape(1, D), W2_0, b2_0.reshape(1, D)),
        (W1_1, b1_1.reshape(1, D), W2_1, b2_1.reshape(1, D)),
        (W1_2, b1_2.reshape(1, D), W2_2, b2_2.reshape(1, D)),
    ]
    for layer, (W1, b1, W2, b2) in enumerate(params):
        agg = _sc_agg(h, src_p, dst_p)
        if layer < 2:
            h = _mlp(h, agg, W1, b1, W2, b2)
        else:
            return _mlp_pool(h, agg, W1, b1, W2, b2, bt)


# speedup vs baseline: 4.1130x; 4.1130x over previous
import functools

import jax
import jax.numpy as jnp
from jax import lax
from jax.experimental import pallas as pl
from jax.experimental.pallas import tpu as pltpu
from jax.experimental.pallas import tpu_sc as plsc

N = 10000
D = 128
E = 320000
G = 64

NC = 2
NS = 16
NW = NC * NS
CH = 128
IB = 16
NB = 5
KPW = NB * IB
EPW = KPW * CH
E_PAD = NW * EPW
N_PAD = 10240
N_CHUNKS = N_PAD // CH
RC_SUB = N_CHUNKS // NS

_mesh = plsc.VectorSubcoreMesh(core_axis_name="c", subcore_axis_name="s")


@functools.partial(
    pl.kernel,
    out_type=jax.ShapeDtypeStruct((NC, N_PAD, D), jnp.float32),
    mesh=_mesh,
    scratch_types=[
        pltpu.VMEM((2, IB, CH), jnp.int32),
        pltpu.VMEM((2, IB, CH), jnp.int32),
        pltpu.VMEM((CH, D), jnp.float32),
        pltpu.VMEM((CH, D), jnp.float32),
        pltpu.VMEM_SHARED((N_PAD, D), jnp.float32),
        pltpu.SemaphoreType.DMA,
        pltpu.SemaphoreType.DMA,
        pltpu.SemaphoreType.DMA,
    ],
)
def _sc_agg(h_hbm, src_hbm, dst_hbm, out_hbm, src_v, dst_v, rows_a, rows_b,
            agg_sh, sem_a, sem_b, sem_i):
    cid = lax.axis_index("c")
    sid = lax.axis_index("s")
    wid = cid * NS + sid

    pltpu.async_copy(src_hbm.at[wid, 0], src_v.at[0], sem_i)
    pltpu.async_copy(dst_hbm.at[wid, 0], dst_v.at[0], sem_i)

    def _zero_row(i, carry):
        for j in range(D // 16):
            rows_a[i, pl.ds(j * 16, 16)] = jnp.zeros((16,), jnp.float32)
        return carry

    lax.fori_loop(0, CH, _zero_row, 0)

    def _zero_chunk(z, carry):
        c = sid * RC_SUB + z
        pltpu.sync_copy(rows_a, agg_sh.at[pl.ds(c * CH, CH)])
        return carry

    lax.fori_loop(0, RC_SUB, _zero_chunk, 0)

    pltpu.make_async_copy(src_hbm.at[wid, 0], src_v.at[0], sem_i).wait()
    pltpu.make_async_copy(dst_hbm.at[wid, 0], dst_v.at[0], sem_i).wait()

    plsc.subcore_barrier()

    for b in range(NB):
        slot = b % 2
        if b + 1 < NB:
            pltpu.async_copy(src_hbm.at[wid, b + 1], src_v.at[1 - slot], sem_i)
            pltpu.async_copy(dst_hbm.at[wid, b + 1], dst_v.at[1 - slot], sem_i)
        pltpu.async_copy(h_hbm.at[src_v.at[slot, 0]], rows_a, sem_a)

        def _pair(p, carry, slot=slot):
            j = 2 * p
            pltpu.async_copy(h_hbm.at[src_v.at[slot, j + 1]], rows_b, sem_b)
            pltpu.make_async_copy(h_hbm.at[src_v.at[slot, j]], rows_a, sem_a).wait()
            pltpu.sync_copy(rows_a, agg_sh.at[dst_v.at[slot, j]], add=True)
            jn = lax.rem(j + 2, IB)
            pltpu.async_copy(h_hbm.at[src_v.at[slot, jn]], rows_a, sem_a)
            pltpu.make_async_copy(h_hbm.at[src_v.at[slot, j + 1]], rows_b, sem_b).wait()
            pltpu.sync_copy(rows_b, agg_sh.at[dst_v.at[slot, j + 1]], add=True)
            return carry

        lax.fori_loop(0, IB // 2, _pair, 0)
        pltpu.make_async_copy(h_hbm.at[src_v.at[slot, 0]], rows_a, sem_a).wait()
        if b + 1 < NB:
            pltpu.make_async_copy(src_hbm.at[wid, b + 1], src_v.at[1 - slot], sem_i).wait()
            pltpu.make_async_copy(dst_hbm.at[wid, b + 1], dst_v.at[1 - slot], sem_i).wait()

    plsc.subcore_barrier()

    def _export_chunk(z, carry):
        c = sid * RC_SUB + z
        pltpu.sync_copy(agg_sh.at[pl.ds(c * CH, CH)], rows_a)
        pltpu.sync_copy(rows_a, out_hbm.at[cid, pl.ds(c * CH, CH)])
        return carry

    lax.fori_loop(0, RC_SUB, _export_chunk, 0)


BLK = 1024


def _mlp_body(h_ref, a_ref, w1_ref, b1_ref, w2_ref, b2_ref, o_ref):
    z = h_ref[...] + a_ref[0] + a_ref[1]
    z = jnp.maximum(
        jnp.dot(z, w1_ref[...], preferred_element_type=jnp.float32) + b1_ref[...],
        0.0,
    )
    z = jnp.dot(z, w2_ref[...], preferred_element_type=jnp.float32) + b2_ref[...]
    o_ref[...] = jnp.maximum(z, 0.0)


_mlp = pl.pallas_call(
    _mlp_body,
    grid=(N_PAD // BLK,),
    in_specs=[
        pl.BlockSpec((BLK, D), lambda i: (i, 0)),
        pl.BlockSpec((NC, BLK, D), lambda i: (0, i, 0)),
        pl.BlockSpec((D, D), lambda i: (0, 0)),
        pl.BlockSpec((1, D), lambda i: (0, 0)),
        pl.BlockSpec((D, D), lambda i: (0, 0)),
        pl.BlockSpec((1, D), lambda i: (0, 0)),
    ],
    out_specs=pl.BlockSpec((BLK, D), lambda i: (i, 0)),
    out_shape=jax.ShapeDtypeStruct((N_PAD, D), jnp.float32),
)


def _mlp_pool_body(h_ref, a_ref, w1_ref, b1_ref, w2_ref, b2_ref, bt_ref, o_ref):
    i = pl.program_id(0)
    z = h_ref[...] + a_ref[0] + a_ref[1]
    z = jnp.maximum(
        jnp.dot(z, w1_ref[...], preferred_element_type=jnp.float32) + b1_ref[...],
        0.0,
    )
    z = jnp.dot(z, w2_ref[...], preferred_element_type=jnp.float32) + b2_ref[...]
    z = jnp.maximum(z, 0.0)
    onehot = (lax.broadcasted_iota(jnp.int32, (BLK, G), 1) == bt_ref[...]).astype(
        jnp.float32
    )
    contrib = lax.dot_general(
        onehot, z, (((0,), (0,)), ((), ())), preferred_element_type=jnp.float32
    )

    @pl.when(i == 0)
    def _():
        o_ref[...] = jnp.zeros_like(o_ref)

    o_ref[...] += contrib


_mlp_pool = pl.pallas_call(
    _mlp_pool_body,
    grid=(N_PAD // BLK,),
    in_specs=[
        pl.BlockSpec((BLK, D), lambda i: (i, 0)),
        pl.BlockSpec((NC, BLK, D), lambda i: (0, i, 0)),
        pl.BlockSpec((D, D), lambda i: (0, 0)),
        pl.BlockSpec((1, D), lambda i: (0, 0)),
        pl.BlockSpec((D, D), lambda i: (0, 0)),
        pl.BlockSpec((1, D), lambda i: (0, 0)),
        pl.BlockSpec((BLK, 1), lambda i: (i, 0)),
    ],
    out_specs=pl.BlockSpec((G, D), lambda i: (0, 0)),
    out_shape=jax.ShapeDtypeStruct((G, D), jnp.float32),
)


def kernel(x, edge_index, batch, W1_0, b1_0, W2_0, b2_0, W1_1, b1_1, W2_1, b2_1,
           W1_2, b1_2, W2_2, b2_2):
    src = edge_index[0]
    dst = edge_index[1]
    pad_e = E_PAD - E
    pad_i = jnp.arange(pad_e, dtype=jnp.int32)
    src_p = jnp.concatenate([src, pad_i % N]).reshape(NW, NB, IB, CH)
    dst_p = jnp.concatenate([dst, N + pad_i % (N_PAD - N)]).reshape(NW, NB, IB, CH)
    h = jnp.pad(x, ((0, N_PAD - N), (0, 0)))
    bt = jnp.concatenate([batch, jnp.full((N_PAD - N,), G, jnp.int32)]).reshape(N_PAD, 1)

    params = [
        (W1_0, b1_0.reshape(1, D), W2_0, b2_0.reshape(1, D)),
        (W1_1, b1_1.reshape(1, D), W2_1, b2_1.reshape(1, D)),
        (W1_2, b1_2.reshape(1, D), W2_2, b2_2.reshape(1, D)),
    ]
    for layer, (W1, b1, W2, b2) in enumerate(params):
        agg = _sc_agg(h, src_p, dst_p)
        if layer < 2:
            h = _mlp(h, agg, W1, b1, W2, b2)
        else:
            return _mlp_pool(h, agg, W1, b1, W2, b2, bt)
